# baseline (speedup 1.0000x reference)
import jax
import jax.numpy as jnp
from jax import lax
from jax.experimental import pallas as pl
from jax.experimental.pallas import tpu as pltpu
from jax.experimental.pallas import tpu_sc as plsc

_B, _S, _D = 4, 2048, 2048
_N = 64
_NR = 4
_KS = (16, 8, 8, 8)
_NCH = 2


def _acc_kernel(x_ref, imp_ref, w_ref, acc_ref):
    w16 = w_ref[...].astype(jnp.bfloat16)
    csz = _S // _NCH
    contribs = [None] * _NR
    for c in range(_NCH):
        x = x_ref[0][c * csz:(c + 1) * csz, :].astype(jnp.bfloat16)
        imp = imp_ref[0][c * csz:(c + 1) * csz, :].astype(jnp.bfloat16)
        logits = jax.lax.dot_general(
            x, w16, (((1,), (0,)), ((), ())),
            preferred_element_type=jnp.float32)
        lt = logits.T
        for r in range(_NR):
            lg = lt[r * _N:(r + 1) * _N, :]
            m = jnp.max(lg, axis=0, keepdims=True)
            e = jnp.exp(lg - m)
            esum = jnp.sum(e, axis=0, keepdims=True)
            p16 = (e / esum).astype(jnp.bfloat16)
            contrib = jax.lax.dot_general(
                p16, imp, (((1,), (0,)), ((), ())),
                preferred_element_type=jnp.float32)
            contribs[r] = contrib if contribs[r] is None else contribs[r] + contrib
    for r in range(_NR):
        acc_ref[0, r * _N:(r + 1) * _N, :] = contribs[r]


def _tc_accumulate(x, importance, w_all):
    acc = pl.pallas_call(
        _acc_kernel,
        grid=(_B,),
        in_specs=[
            pl.BlockSpec((1, _S, _D), lambda b: (b, 0, 0)),
            pl.BlockSpec((1, _S, 1), lambda b: (b, 0, 0)),
            pl.BlockSpec((_D, _NR * _N), lambda b: (0, 0)),
        ],
        out_specs=pl.BlockSpec((1, _NR * _N, 1), lambda b: (b, 0, 0)),
        out_shape=jax.ShapeDtypeStruct((_B, _NR * _N, 1), jnp.float32),
        compiler_params=pltpu.CompilerParams(
            dimension_semantics=("arbitrary",)),
    )(x, importance.reshape(_B, _S, 1), w_all)
    return acc.reshape(_B * _NR, _N)


def _sc_epilogue_kernel(acc_hbm, topw_hbm, topi_hbm, maskout_hbm,
                        row_v, wv_v, iv_v, mask_v):
    wid = lax.axis_index("s") * 2 + lax.axis_index("c")

    @pl.when(wid < _B * _NR)
    def _task():
        r = wid % _NR
        pltpu.sync_copy(acc_hbm.at[pl.ds(wid * _N, _N)], row_v)
        iota = lax.iota(jnp.int32, 16)

        def rot(v, s):
            idx = (iota + s) & 15
            return lax.gather(
                v, idx[:, None],
                lax.GatherDimensionNumbers(
                    offset_dims=(), collapsed_slice_dims=(0,),
                    start_index_map=(0,)),
                (1,), mode=lax.GatherScatterMode.PROMISE_IN_BOUNDS)

        def allred(v, op):
            for s in (1, 2, 4, 8):
                v = op(v, rot(v, s))
            return v

        vs = [row_v[pl.ds(16 * q, 16)] for q in range(4)]
        t1 = allred((vs[0] + vs[1]) + (vs[2] + vs[3]), jnp.add) + 1e-8
        vs = [v / t1 for v in vs]
        one = jnp.full((16,), 1, jnp.int32)
        kk = _KS[1] + (_KS[0] - _KS[1]) * (1 - jnp.minimum(r, 1))
        kv = kk * one
        masks = [jnp.zeros((16,), jnp.float32) for _ in range(4)]
        wv = jnp.zeros((16,), jnp.float32)
        iv = jnp.zeros((16,), jnp.int32)
        tsum = jnp.zeros((16,), jnp.float32)
        for j in range(_KS[0]):
            act = jnp.minimum(jnp.maximum(kv - j, 0), 1)
            actf = act.astype(jnp.float32)
            m = allred(jnp.maximum(jnp.maximum(vs[0], vs[1]),
                                   jnp.maximum(vs[2], vs[3])), jnp.maximum)
            cand = jnp.full((16,), 4 * 16, jnp.int32)
            for q in range(4):
                cand = jnp.minimum(cand, jnp.where(
                    vs[q] == m, iota + 16 * q, 4 * 16))
            ix = allred(cand, jnp.minimum)
            tsum = tsum + m * actf
            put = (1 - jnp.minimum(jnp.abs(iota - j), 1)) * act
            putf = put.astype(jnp.float32)
            wv = wv * (1.0 - putf) + m * putf
            iv = iv * (1 - put) + ix * put
            for q in range(4):
                hit = (1 - jnp.minimum(jnp.abs(iota + 16 * q - ix), 1)) * act
                hitf = hit.astype(jnp.float32)
                masks[q] = jnp.maximum(masks[q], hitf)
                vs[q] = vs[q] * (1.0 - hitf) - hitf
        wv_v[pl.ds(0, 16)] = wv * (1.0 / (tsum + 1e-8))
        iv_v[pl.ds(0, 16)] = iv
        for q in range(4):
            mask_v[pl.ds(16 * q, 16)] = masks[q]
        pltpu.sync_copy(wv_v, topw_hbm.at[pl.ds(wid * _N, _N)])
        pltpu.sync_copy(iv_v, topi_hbm.at[pl.ds(wid * _N, _N)])
        pltpu.sync_copy(mask_v, maskout_hbm.at[pl.ds(wid * _N, _N)])


def _sc_epilogue(acc_flat):
    f32, i32 = jnp.float32, jnp.int32
    nt = _B * _NR * _N
    out_type = (
        jax.ShapeDtypeStruct((nt,), f32),
        jax.ShapeDtypeStruct((nt,), i32),
        jax.ShapeDtypeStruct((nt,), f32),
    )
    fn = pl.kernel(
        _sc_epilogue_kernel,
        out_type=out_type,
        mesh=plsc.VectorSubcoreMesh(core_axis_name="c", subcore_axis_name="s"),
        scratch_types=[
            pltpu.VMEM((_N,), f32),
            pltpu.VMEM((_N,), f32),
            pltpu.VMEM((_N,), i32),
            pltpu.VMEM((_N,), f32),
        ],
    )
    return fn(acc_flat)


def kernel(x, importance, Wc, WQ, WK, WV):
    w_all = jnp.concatenate([Wc, WQ, WK, WV], axis=1)
    acc = _tc_accumulate(x, importance, w_all)
    topw, topi, maskout = _sc_epilogue(acc.reshape(-1))
    tw = topw.reshape(_B, _NR, _N)
    ti = topi.reshape(_B, _NR, _N)
    mo = maskout.reshape(_B, _NR, _N)
    return (tw[:, 0, :_KS[0]], ti[:, 0, :_KS[0]],
            tw[:, 1, :_KS[1]], ti[:, 1, :_KS[1]],
            tw[:, 2, :_KS[2]], ti[:, 2, :_KS[2]],
            tw[:, 3, :_KS[3]], ti[:, 3, :_KS[3]],
            mo[:, 0], mo[:, 1], mo[:, 2], mo[:, 3])
